# baseline (speedup 1.0000x reference)
import functools

import jax
import jax.numpy as jnp
from jax import lax
from jax.experimental import pallas as pl
from jax.experimental.pallas import tpu as pltpu
from jax.experimental.pallas import tpu_sc as plsc

SEQ = 8192
DIM = 2048
TOT = 4 * 8192
NC, NS = 2, 16
NW = NC * NS
PER_W = TOT // NW
NBUF = 2
CHUNK = 16
NCHUNK = PER_W // CHUNK
NGROUP = NCHUNK // NBUF

_mesh = plsc.VectorSubcoreMesh(core_axis_name="c", subcore_axis_name="s")


@functools.partial(
    pl.kernel,
    out_type=jax.ShapeDtypeStruct((TOT, DIM), jnp.float32),
    mesh=_mesh,
    scratch_types=[
        pltpu.VMEM((PER_W,), jnp.int32),
        pltpu.VMEM_SHARED((NS * NBUF * CHUNK, DIM), jnp.float32),
        [pltpu.SemaphoreType.DMA] * NBUF,
        [pltpu.SemaphoreType.DMA] * NBUF,
    ],
)
def _gather_sc(ids_hbm, table_hbm, out_hbm, idx_v, stage, gsems, psems):
    wid = lax.axis_index("s") * NC + lax.axis_index("c")
    sid = lax.axis_index("s")
    base = wid * PER_W

    pltpu.sync_copy(ids_hbm.at[wid], idx_v)

    def buf(b):
        return stage.at[pl.ds((sid * NBUF + b) * CHUNK, CHUNK)]

    def gather(j, b):
        vec = idx_v[pl.ds(j * CHUNK, CHUNK)]
        slab = buf(b)
        for k in range(CHUNK):
            pltpu.async_copy(
                table_hbm.at[pl.ds(vec[k], 1)], slab.at[pl.ds(k, 1)], gsems[b]
            )

    def gwait(b):
        slab = buf(b)
        for k in range(CHUNK):
            pltpu.make_async_copy(
                table_hbm.at[pl.ds(0, 1)], slab.at[pl.ds(k, 1)], gsems[b]
            ).wait()

    def put(j, b):
        dst = out_hbm.at[pl.ds(base + j * CHUNK, CHUNK)]
        pltpu.async_copy(buf(b), dst, psems[b])

    def pwait(b):
        dst = out_hbm.at[pl.ds(base, CHUNK)]
        pltpu.make_async_copy(buf(b), dst, psems[b]).wait()

    for b in range(NBUF):
        gather(b, b)

    def body(g, carry):
        j0 = g * NBUF
        for b in range(NBUF):
            gwait(b)
            put(j0 + b, b)
        for b in range(NBUF):
            pwait(b)
            gather(j0 + NBUF + b, b)
        return carry

    lax.fori_loop(0, NGROUP - 1, body, 0)

    j0 = (NGROUP - 1) * NBUF
    for b in range(NBUF):
        gwait(b)
        put(j0 + b, b)
    for b in range(NBUF):
        pwait(b)


def kernel(position_ids, table):
    ids = position_ids.reshape(NW, PER_W).astype(jnp.int32)
    out = _gather_sc(ids, table)
    return out.reshape(position_ids.shape[0], position_ids.shape[1], DIM)
